# Initial kernel scaffold; baseline (speedup 1.0000x reference)
import jax
import jax.numpy as jnp
from jax.experimental import pallas as pl


def kernel(z, codebook):
    raise NotImplementedError("write your pallas kernel here")


# speedup vs baseline: 2.7521x; 2.7521x over previous
import jax
import jax.numpy as jnp
from jax.experimental import pallas as pl

_N = 1024
_K = 64
_TB = 512


def _vq_block(x_ref, cb_ref, o_ref):
    x = x_ref[...]
    cb = cb_ref[...]
    sim = jnp.dot(x, cb, preferred_element_type=jnp.float32)
    xsq = jnp.sum(x * x, axis=1, keepdims=True)
    csq = jnp.sum(cb * cb, axis=0, keepdims=True)
    dist = xsq + csq - 2.0 * sim
    m = jnp.min(dist, axis=1, keepdims=True)
    ids = jax.lax.broadcasted_iota(jnp.int32, (_TB, _N), 1)
    idx = jnp.min(jnp.where(dist == m, ids, _N), axis=1)
    oh = (ids == idx[:, None]).astype(jnp.float32)
    q = jax.lax.dot_general(oh, cb, (((1,), (1,)), ((), ())),
                            preferred_element_type=jnp.float32)
    o_ref[...] = x + (q - x)


def kernel(z, codebook):
    shape = z.shape
    flat = z.reshape(-1, _K)
    t = flat.shape[0]
    grid = t // _TB
    out = pl.pallas_call(
        _vq_block,
        grid=(grid,),
        in_specs=[
            pl.BlockSpec((_TB, _K), lambda i: (i, 0)),
            pl.BlockSpec((_K, _N), lambda i: (0, 0)),
        ],
        out_specs=pl.BlockSpec((_TB, _K), lambda i: (i, 0)),
        out_shape=jax.ShapeDtypeStruct((t, _K), jnp.float32),
    )(flat, codebook)
    return out.reshape(shape)
